# baseline (speedup 1.0000x reference)
import functools

import jax
import jax.numpy as jnp
from jax.experimental import pallas as pl
from jax.experimental.pallas import tpu as pltpu

NE = 8
TOPK = 2
D = 2048
N = 2048
T = 256
P = ((N * TOPK + NE * (T - 1) + T - 1) // T) * T
GT = P // T


def _mm1_body(sp_ref, x_ref, w1_ref, o_ref):
    i = pl.program_id(0)

    @pl.when(i < sp_ref[GT])
    def _():
        h = jnp.dot(x_ref[...], w1_ref[0], preferred_element_type=jnp.float32)
        o_ref[...] = 0.5 * h * (1.0 + jax.lax.erf(h * 0.7071067811865476))


def _mm2_body(sp_ref, h_ref, w2_ref, o_ref):
    i = pl.program_id(0)

    @pl.when(i < sp_ref[GT])
    def _():
        o_ref[...] = jnp.dot(h_ref[...], w2_ref[0],
                             preferred_element_type=jnp.float32)


def _grouped_ffn(x_sorted, w1, w2, e_of_tile, nvalid):
    sp = jnp.concatenate([e_of_tile, nvalid[None]]).astype(jnp.int32)

    def call(body, x, w):
        grid_spec = pltpu.PrefetchScalarGridSpec(
            num_scalar_prefetch=1,
            grid=(GT,),
            in_specs=[
                pl.BlockSpec((T, D), lambda i, sp: (i, 0)),
                pl.BlockSpec((1, D, D), lambda i, sp: (sp[i], 0, 0)),
            ],
            out_specs=pl.BlockSpec((T, D), lambda i, sp: (i, 0)),
        )
        return pl.pallas_call(
            body,
            grid_spec=grid_spec,
            out_shape=jax.ShapeDtypeStruct((P, D), jnp.float32),
        )(sp, x, w)

    h_sorted = call(_mm1_body, x_sorted, w1)
    return call(_mm2_body, h_sorted, w2)


def kernel(tokens, router_w, w1, w2):
    i32 = jnp.int32
    scores = jax.nn.softmax(tokens @ router_w.T, axis=-1)
    topw, topi = jax.lax.top_k(scores, TOPK)

    e_flat = topi.reshape(-1).astype(i32)
    onehot = (e_flat[:, None] == jnp.arange(NE, dtype=i32)[None, :]).astype(i32)
    cnt_inc = jnp.cumsum(onehot, axis=0)
    counts = cnt_inc[-1]
    rank = jnp.take_along_axis(cnt_inc, e_flat[:, None], axis=1)[:, 0] - 1
    pc = ((counts + T - 1) // T) * T
    cum_pc = jnp.cumsum(pc)
    po = cum_pc - pc
    pos = po[e_flat] + rank
    nvalid = (cum_pc[-1] // T).astype(i32)

    tok_of_pos = jnp.zeros((P,), i32).at[pos].set(jnp.arange(N * TOPK, dtype=i32) // TOPK)

    tile_start = jnp.arange(GT, dtype=i32) * T
    e_of_tile = jnp.minimum(
        jnp.searchsorted(cum_pc, tile_start, side="right").astype(i32), NE - 1)
    e_last = e_of_tile[jnp.maximum(nvalid - 1, 0)]
    e_of_tile = jnp.where(jnp.arange(GT, dtype=i32) < nvalid, e_of_tile, e_last)

    x_sorted = tokens[tok_of_pos]

    y_sorted = _grouped_ffn(x_sorted, w1, w2, e_of_tile, nvalid)

    ps = pos.reshape(N, TOPK)
    out = (y_sorted[ps[:, 0]] * topw[:, 0:1]
           + y_sorted[ps[:, 1]] * topw[:, 1:2])
    return out
